# baseline (speedup 1.0000x reference)
import functools

import jax
import jax.numpy as jnp
from jax import lax
from jax.experimental import pallas as pl
from jax.experimental.pallas import tpu as pltpu
from jax.experimental.pallas import tpu_sc as plsc

_LANES = 16


@functools.lru_cache(maxsize=None)
def _make_sc_kernel(B, C, HH, WW, L):
    HALF = C // 2
    CH = _LANES
    NK = HALF // CH
    RPC = CH * HH
    mesh = plsc.VectorSubcoreMesh(core_axis_name="c", subcore_axis_name="s")

    @functools.partial(
        pl.kernel,
        out_type=jax.ShapeDtypeStruct((B, 2 * C, HH, WW), jnp.float32),
        mesh=mesh,
        scratch_types=[
            pltpu.VMEM((L,), jnp.int32),
            pltpu.VMEM((L, C), jnp.float32),
            pltpu.VMEM((C,), jnp.float32),
            pltpu.VMEM((3, RPC, WW), jnp.float32),
            pltpu.VMEM((2, RPC, WW), jnp.float32),
            pltpu.SemaphoreType.DMA,
            pltpu.SemaphoreType.DMA((3,)),
            pltpu.SemaphoreType.DMA((3,)),
            pltpu.SemaphoreType.DMA((2,)),
        ],
    )
    def sc_kernel(idx_hbm, w_hbm, lat4_hbm, out4_hbm,
                  idx_v, rows_v, aux_v, lbuf, cbuf,
                  gsem, isems, osems, csems):
        lat_hbm = lat4_hbm.reshape(B * C * HH, WW)
        out_hbm = out4_hbm.reshape(B * 2 * C * HH, WW)
        wid = lax.axis_index("s") * 2 + lax.axis_index("c")
        b = wid // 2
        c0 = (wid % 2) * HALF

        pltpu.sync_copy(idx_hbm.at[b], idx_v)
        pltpu.async_copy(w_hbm.at[idx_v], rows_v, gsem).wait()

        def asum(j, carry):
            sl = pl.ds(j * _LANES, _LANES)
            acc = rows_v[0, sl]
            for l in range(1, L):
                acc = acc + rows_v[l, sl]
            aux_v[sl] = acc
            return carry

        lax.fori_loop(0, C // _LANES, asum, 0)

        def in_b(k):
            r0 = (b * C + c0 + k * CH) * HH
            return pltpu.make_async_copy(
                lat_hbm.at[pl.ds(r0, RPC)], lbuf.at[k % 3], isems.at[k % 3])

        def out_b(k):
            r0 = (b * 2 * C + c0 + k * CH) * HH
            return pltpu.make_async_copy(
                lbuf.at[k % 3], out_hbm.at[pl.ds(r0, RPC)], osems.at[k % 3])

        def out_c(k):
            r0 = (b * 2 * C + C + c0 + k * CH) * HH
            return pltpu.make_async_copy(
                cbuf.at[k % 2], out_hbm.at[pl.ds(r0, RPC)], csems.at[k % 2])

        def fill_c(k):
            bi = k % 2
            grp = aux_v[pl.ds(c0 + k * CH, CH)]
            for j in range(CH):
                v = jnp.full((_LANES,), grp[j], jnp.float32)
                for r in range(HH):
                    for g in range(WW // _LANES):
                        cbuf[bi, j * HH + r, pl.ds(g * _LANES, _LANES)] = v

        in_b(0).start()
        in_b(1).start()
        in_b(0).wait()
        out_b(0).start()
        fill_c(0)
        out_c(0).start()
        in_b(2).start()
        in_b(1).wait()
        out_b(1).start()
        fill_c(1)
        out_c(1).start()

        def body(k, carry):
            out_b(k - 2).wait()
            out_c(k - 2).wait()
            in_b(k + 1).start()
            in_b(k).wait()
            out_b(k).start()
            fill_c(k)
            out_c(k).start()
            return carry

        lax.fori_loop(2, NK - 1, body, 0)

        out_b(NK - 3).wait()
        out_c(NK - 3).wait()
        in_b(NK - 1).wait()
        out_b(NK - 1).start()
        fill_c(NK - 1)
        out_c(NK - 1).start()

        out_b(NK - 2).wait()
        out_c(NK - 2).wait()
        out_b(NK - 1).wait()
        out_c(NK - 1).wait()

    return sc_kernel


def kernel(latents, msg, W_emb):
    B, C, H, W = latents.shape
    L = msg.shape[-1]
    msg_i = msg.astype(jnp.int32)
    idx = (2 * jnp.arange(L, dtype=jnp.int32))[None, :] + msg_i
    out = _make_sc_kernel(B, C, 8, 128, L)(
        idx, W_emb, latents.reshape(B, C, 8, 128))
    return out.reshape(B, 2 * C, H, W)
